# baseline (speedup 1.0000x reference)
import jax
import jax.numpy as jnp
from jax.experimental import pallas as pl

_NB, _NA, _ATTR = 16, 3, 85
_NH = _NW = 76
_NC = _NA * _ATTR
_STRIDE = 8.0
_AW = (116.0, 156.0, 373.0)
_AH = (90.0, 198.0, 326.0)

_TH = 16
_NTILES = (_NH + _TH - 1) // _TH
_TP = _TH * _NW


def _body(x_ref, o_ref):
    j = pl.program_id(1)
    v = x_ref[0].reshape(_NC, _TP)
    t = v.T
    c = jax.lax.broadcasted_iota(jnp.int32, (1, _NC), 1)
    r = c % _ATTR
    a = c // _ATTR
    isexp = (r == 2) | (r == 3)
    e = jnp.exp(jnp.where(isexp, t, -t))
    base = jnp.where(isexp, e, 1.0 / (1.0 + e))
    aw = jnp.where(a == 0, _AW[0], jnp.where(a == 1, _AW[1], _AW[2]))
    ah = jnp.where(a == 0, _AH[0], jnp.where(a == 1, _AH[1], _AH[2]))
    mul = jnp.where(r < 2, _STRIDE,
          jnp.where(r == 2, aw,
          jnp.where(r == 3, ah, 1.0))).astype(jnp.float32)
    p = j * _TP + jax.lax.broadcasted_iota(jnp.int32, (_TP, 1), 0)
    w = (p % _NW).astype(jnp.float32)
    h = (p // _NW).astype(jnp.float32)
    m0 = (r == 0).astype(jnp.float32)
    m1 = (r == 1).astype(jnp.float32)
    add = m0 * (_STRIDE * w) + m1 * (_STRIDE * h)
    o_ref[0] = base * mul + add


def kernel(x):
    out = pl.pallas_call(
        _body,
        grid=(_NB, _NTILES),
        in_specs=[pl.BlockSpec((1, _NC, _TH, _NW), lambda b, j: (b, 0, j, 0))],
        out_specs=pl.BlockSpec((1, _TP, _NC), lambda b, j: (b, j, 0)),
        out_shape=jax.ShapeDtypeStruct((_NB, _NH * _NW, _NC), jnp.float32),
    )(x)
    return out.reshape(_NB, _NA * _NH * _NW, _ATTR)


# speedup vs baseline: 1.2461x; 1.2461x over previous
import jax
import jax.numpy as jnp
from jax.experimental import pallas as pl

_NB, _NA, _ATTR = 16, 3, 85
_NH = _NW = 76
_NP = _NH * _NW
_NC = _NA * _ATTR
_STRIDE = 8.0
_AW = (116.0, 156.0, 373.0)
_AH = (90.0, 198.0, 326.0)

_TP = _NP
_NTILES = _NP // _TP


def _body(x_ref, o_ref):
    j = pl.program_id(1)
    v = x_ref[0]
    t = v.T
    c = jax.lax.broadcasted_iota(jnp.int32, (1, _NC), 1)
    r = c % _ATTR
    a = c // _ATTR
    isexp = (r == 2) | (r == 3)
    e = jnp.exp(jnp.where(isexp, t, -t))
    base = jnp.where(isexp, e, 1.0 / (1.0 + e))
    aw = jnp.where(a == 0, _AW[0], jnp.where(a == 1, _AW[1], _AW[2]))
    ah = jnp.where(a == 0, _AH[0], jnp.where(a == 1, _AH[1], _AH[2]))
    mul = jnp.where(r < 2, _STRIDE,
          jnp.where(r == 2, aw,
          jnp.where(r == 3, ah, 1.0))).astype(jnp.float32)
    p = j * _TP + jax.lax.broadcasted_iota(jnp.int32, (_TP, 1), 0)
    w = (p % _NW).astype(jnp.float32)
    h = (p // _NW).astype(jnp.float32)
    m0 = (r == 0).astype(jnp.float32)
    m1 = (r == 1).astype(jnp.float32)
    add = m0 * (_STRIDE * w) + m1 * (_STRIDE * h)
    o_ref[0] = base * mul + add


def kernel(x):
    xr = x.reshape(_NB, _NC, _NP)
    out = pl.pallas_call(
        _body,
        grid=(_NB, _NTILES),
        in_specs=[pl.BlockSpec((1, _NC, _TP), lambda b, j: (b, 0, j))],
        out_specs=pl.BlockSpec((1, _TP, _NC), lambda b, j: (b, j, 0)),
        out_shape=jax.ShapeDtypeStruct((_NB, _NP, _NC), jnp.float32),
    )(xr)
    return out.reshape(_NB, _NA * _NP, _ATTR)


# speedup vs baseline: 1.6727x; 1.3424x over previous
import jax
import jax.numpy as jnp
from jax.experimental import pallas as pl

_NB, _NA, _ATTR = 16, 3, 85
_NH = _NW = 76
_NP = _NH * _NW
_NC = _NA * _ATTR
_STRIDE = 8.0
_AW = (116.0, 156.0, 373.0)
_AH = (90.0, 198.0, 326.0)

_TP = _NP
_NTILES = _NP // _TP


def _body(x_ref, o_ref):
    j = pl.program_id(1)
    v = x_ref[0]
    t = v.T
    c = jax.lax.broadcasted_iota(jnp.int32, (1, _NC), 1)
    r = c % _ATTR
    a = c // _ATTR
    isexp = (r == 2) | (r == 3)
    e = jnp.exp(jnp.where(isexp, t, -t))
    base = jnp.where(isexp, e, 1.0 / (1.0 + e))
    aw = jnp.where(a == 0, _AW[0], jnp.where(a == 1, _AW[1], _AW[2]))
    ah = jnp.where(a == 0, _AH[0], jnp.where(a == 1, _AH[1], _AH[2]))
    mul = jnp.where(r < 2, _STRIDE,
          jnp.where(r == 2, aw,
          jnp.where(r == 3, ah, 1.0))).astype(jnp.float32)
    p = j * _TP + jax.lax.broadcasted_iota(jnp.int32, (_TP, 1), 0)
    w = (p % _NW).astype(jnp.float32)
    h = (p // _NW).astype(jnp.float32)
    m0 = (r == 0).astype(jnp.float32)
    m1 = (r == 1).astype(jnp.float32)
    add = m0 * (_STRIDE * w) + m1 * (_STRIDE * h)
    res = base * mul + add
    for anc in range(_NA):
        o_ref[0, pl.Slice(anc, _TP, _NA), :] = res[:, anc * _ATTR:(anc + 1) * _ATTR]


def kernel(x):
    xr = x.reshape(_NB, _NC, _NP)
    return pl.pallas_call(
        _body,
        grid=(_NB, _NTILES),
        in_specs=[pl.BlockSpec((1, _NC, _TP), lambda b, j: (b, 0, j))],
        out_specs=pl.BlockSpec((1, _TP * _NA, _ATTR), lambda b, j: (b, j, 0)),
        out_shape=jax.ShapeDtypeStruct((_NB, _NP * _NA, _ATTR), jnp.float32),
    )(xr)
